# baseline (speedup 1.0000x reference)
import jax
import jax.numpy as jnp
from jax.experimental import pallas as pl
from jax.experimental.pallas import tpu as pltpu

_NSEM = 8


def _body(w_ref, sc_hbm, out_hbm, wt_ref, w_sems, s_sems):
    g = sc_hbm.shape[0]
    d = w_ref.shape[1]

    wt_ref[...] = jnp.transpose(w_ref[...], (1, 0))

    def issue(i, _):
        s = jax.lax.rem(i, _NSEM)
        pltpu.make_async_copy(
            wt_ref, out_hbm.at[i, pl.ds(0, d), :], w_sems.at[s]
        ).start()
        pltpu.make_async_copy(
            sc_hbm.at[pl.ds(i, 1), :], out_hbm.at[i, pl.ds(d, 1), :], s_sems.at[s]
        ).start()
        return ()

    def drain(i, _):
        s = jax.lax.rem(i, _NSEM)
        pltpu.make_async_copy(
            wt_ref, out_hbm.at[i, pl.ds(0, d), :], w_sems.at[s]
        ).wait()
        pltpu.make_async_copy(
            sc_hbm.at[pl.ds(i, 1), :], out_hbm.at[i, pl.ds(d, 1), :], s_sems.at[s]
        ).wait()
        return ()

    jax.lax.fori_loop(0, g, issue, ())
    jax.lax.fori_loop(0, g, drain, ())


def kernel(scRNA_count, embedding_weight):
    g, c = scRNA_count.shape
    c2, d = embedding_weight.shape
    assert c2 == c

    return pl.pallas_call(
        _body,
        in_specs=[
            pl.BlockSpec(memory_space=pltpu.VMEM),
            pl.BlockSpec(memory_space=pltpu.MemorySpace.HBM),
        ],
        out_specs=pl.BlockSpec(memory_space=pltpu.MemorySpace.HBM),
        out_shape=jax.ShapeDtypeStruct((g, d + 1, c), jnp.float32),
        scratch_shapes=[
            pltpu.VMEM((d, c), jnp.float32),
            pltpu.SemaphoreType.DMA((_NSEM,)),
            pltpu.SemaphoreType.DMA((_NSEM,)),
        ],
    )(embedding_weight, scRNA_count)


# speedup vs baseline: 1.4604x; 1.4604x over previous
import jax
import jax.numpy as jnp
from jax.experimental import pallas as pl
from jax.experimental.pallas import tpu as pltpu


def _body(w_ref, sc_ref, out_ref, wt_ref):
    gblk = out_ref.shape[0]
    d = w_ref.shape[1]
    c = w_ref.shape[0]

    @pl.when(pl.program_id(0) == 0)
    def _():
        wt_ref[...] = jnp.transpose(w_ref[...], (1, 0))

    wt = wt_ref[...]
    out_ref[:, :d, :] = jnp.broadcast_to(wt[None, :, :], (gblk, d, c))
    out_ref[:, d:, :] = sc_ref[...][:, None, :]


def kernel(scRNA_count, embedding_weight):
    g, c = scRNA_count.shape
    c2, d = embedding_weight.shape
    assert c2 == c
    gblk = 16

    return pl.pallas_call(
        _body,
        grid=(g // gblk,),
        in_specs=[
            pl.BlockSpec((c, d), lambda i: (0, 0)),
            pl.BlockSpec((gblk, c), lambda i: (i, 0)),
        ],
        out_specs=pl.BlockSpec((gblk, d + 1, c), lambda i: (i, 0, 0)),
        out_shape=jax.ShapeDtypeStruct((g, d + 1, c), jnp.float32),
        scratch_shapes=[pltpu.VMEM((d, c), jnp.float32)],
    )(embedding_weight, scRNA_count)
